# baseline (speedup 1.0000x reference)
import functools

import jax
import jax.numpy as jnp
from jax import lax
from jax.experimental import pallas as pl
from jax.experimental.pallas import tpu as pltpu
from jax.experimental.pallas import tpu_sc as plsc

_NC = 2
_NS = 16
_LANES = 16


def _build(F, B, L, V, D):
    NW = _NC * _NS
    NB = B // NW

    mesh = plsc.VectorSubcoreMesh(
        core_axis_name="c", subcore_axis_name="s",
        num_cores=_NC, num_subcores=_NS)

    @functools.partial(
        pl.kernel,
        out_type=jax.ShapeDtypeStruct((B, F * D), jnp.float32),
        mesh=mesh,
        compiler_params=pltpu.CompilerParams(use_tc_tiling_on_sc=False, needs_layout_passes=False),
        scratch_types=[
            pltpu.VMEM((NB, L), jnp.int32),
            pltpu.VMEM((NB * L, D), jnp.float32),
            pltpu.VMEM((NB, D), jnp.float32),
            pltpu.VMEM((F, L), jnp.float32),
            pltpu.SemaphoreType.DMA,
        ],
    )
    def run(idx_hbm, table_hbm, pw_hbm, out_hbm, idx_v, rows_v, out_v, pw_v,
            sem):
        wid = lax.axis_index("s") * _NC + lax.axis_index("c")
        pltpu.sync_copy(pw_hbm, pw_v)
        lanes0 = jnp.zeros((_LANES,), jnp.int32)

        def f_body(f, _):
            pltpu.sync_copy(idx_hbm.at[f, pl.ds(wid * NB, NB), :], idx_v)

            def gfire(g, _):
                for i in range(16):
                    b = g * 16 + i
                    pltpu.async_copy(table_hbm.at[idx_v.at[b]],
                                     rows_v.at[pl.ds(b * L, L)], sem)
                return 0

            lax.fori_loop(0, NB // 16, gfire, 0)
            pltpu.make_async_copy(table_hbm.at[pl.ds(0, NB * L)],
                                  rows_v, sem).wait()
            wv = [
                plsc.load_gather(pw_v, [lanes0 + f, lanes0 + l])
                for l in range(L)
            ]

            def bag(i, _):
                base = i * L
                acc0 = jnp.zeros((_LANES,), jnp.float32)
                acc1 = jnp.zeros((_LANES,), jnp.float32)
                for l in range(L):
                    acc0 = acc0 + wv[l] * rows_v[base + l, 0:16]
                    acc1 = acc1 + wv[l] * rows_v[base + l, 16:32]
                out_v[i, 0:16] = acc0
                out_v[i, 16:32] = acc1
                return 0

            lax.fori_loop(0, NB, bag, 0)
            pltpu.sync_copy(out_v, out_hbm.at[pl.ds(wid * NB, NB),
                                              pl.ds(f * D, D)])
            return 0

        lax.fori_loop(0, F, f_body, 0)

    return run


def kernel(indices, table, pos_weight):
    F, B, L = indices.shape
    V, D = table.shape
    run = _build(F, B, L, V, D)
    return run(indices.astype(jnp.int32), table,
               pos_weight.astype(jnp.float32))


# speedup vs baseline: 1.0918x; 1.0918x over previous
import functools

import jax
import jax.numpy as jnp
from jax import lax
from jax.experimental import pallas as pl
from jax.experimental.pallas import tpu as pltpu
from jax.experimental.pallas import tpu_sc as plsc

_NC = 2
_NS = 16
_LANES = 16


def _build(F, B, L, V, D):
    NW = _NC * _NS
    NB = B // NW

    mesh = plsc.VectorSubcoreMesh(
        core_axis_name="c", subcore_axis_name="s",
        num_cores=_NC, num_subcores=_NS)

    @functools.partial(
        pl.kernel,
        out_type=jax.ShapeDtypeStruct((B, F * D), jnp.float32),
        mesh=mesh,
        compiler_params=pltpu.CompilerParams(use_tc_tiling_on_sc=False),
        scratch_types=[
            pltpu.VMEM((L, NB), jnp.int32),
            pltpu.VMEM((L * NB, D), jnp.float32),
            pltpu.VMEM((NB, D), jnp.float32),
            pltpu.VMEM((F, L, _LANES), jnp.float32),
            pltpu.SemaphoreType.DMA,
        ],
    )
    def run(idx_hbm, table_hbm, pwe_hbm, out_hbm, idx_v, rows_v, out_v, pw_v,
            sem):
        wid = lax.axis_index("s") * _NC + lax.axis_index("c")
        pltpu.sync_copy(pwe_hbm, pw_v)

        def f_body(f, _):
            pltpu.sync_copy(idx_hbm.at[f, :, pl.ds(wid * NB, NB)], idx_v)
            cps = [
                pltpu.async_copy(table_hbm.at[idx_v.at[l]],
                                 rows_v.at[pl.ds(l * NB, NB)], sem)
                for l in range(L)
            ]
            for c in cps:
                c.wait()
            wv = [pw_v[f, l, :] for l in range(L)]

            def bag(i, _):
                acc0 = jnp.zeros((_LANES,), jnp.float32)
                acc1 = jnp.zeros((_LANES,), jnp.float32)
                for l in range(L):
                    acc0 = acc0 + wv[l] * rows_v[l * NB + i, 0:16]
                    acc1 = acc1 + wv[l] * rows_v[l * NB + i, 16:32]
                out_v[i, 0:16] = acc0
                out_v[i, 16:32] = acc1
                return 0

            lax.fori_loop(0, NB, bag, 0)
            pltpu.sync_copy(out_v, out_hbm.at[pl.ds(wid * NB, NB),
                                              pl.ds(f * D, D)])
            return 0

        lax.fori_loop(0, F, f_body, 0)

    return run


def kernel(indices, table, pos_weight):
    F, B, L = indices.shape
    V, D = table.shape
    idx_t = jnp.transpose(indices.astype(jnp.int32), (0, 2, 1))
    pwe = jnp.broadcast_to(
        pos_weight.astype(jnp.float32)[:, :, None], (F, L, _LANES))
    run = _build(F, B, L, V, D)
    return run(idx_t, table, pwe)
